# baseline (speedup 1.0000x reference)
import functools

import jax
import jax.numpy as jnp
from jax import lax
from jax.experimental import pallas as pl
from jax.experimental.pallas import tpu as pltpu
from jax.experimental.pallas import tpu_sc as plsc

N_NODES = 10000
N_EDGES = 320000
D = 128
N_GRAPHS = 64
D_OUT = 16

NC = 2
NS = 16
NW = NC * NS
EC = 128

N_PAD = 10240
N_ROWCH = N_PAD // EC
ROWCH_PER_S = N_ROWCH // NS
E_PAD = 327680
N_CHUNKS = E_PAD // EC
CHUNKS_PER_W = N_CHUNKS // NW
HALF_W = CHUNKS_PER_W // 2

BLK = 2048
GRID = N_PAD // BLK


def _sc_segment_pass(with_deg: bool):
  mesh = plsc.VectorSubcoreMesh(core_axis_name="c", subcore_axis_name="s")
  out_type = [jax.ShapeDtypeStruct((NC, N_PAD, D), jnp.float32)]
  scratch = [
      pltpu.VMEM((HALF_W, EC), jnp.int32),
      pltpu.VMEM((HALF_W, EC), jnp.int32),
      pltpu.VMEM((EC, D), jnp.float32),
      pltpu.VMEM((EC, D), jnp.float32),
      pltpu.VMEM_SHARED((N_PAD, D), jnp.float32),
      pltpu.SemaphoreType.DMA,
      pltpu.SemaphoreType.DMA,
  ]
  if with_deg:
    out_type.append(jax.ShapeDtypeStruct((NC, N_PAD), jnp.float32))
    scratch += [
        pltpu.VMEM((EC,), jnp.float32),
        pltpu.VMEM((EC,), jnp.float32),
        pltpu.VMEM_SHARED((N_PAD,), jnp.float32),
    ]

  def body(y_hbm, src_hbm, dst_hbm, zeros_hbm, ones_hbm, *rest):
    if with_deg:
      (out_hbm, deg_hbm, src_v, dst_v, rows_a, rows_b, acc_sh,
       sem_a, sem_b, ones_v, zcol_v, deg_sh) = rest
    else:
      (out_hbm, src_v, dst_v, rows_a, rows_b, acc_sh, sem_a, sem_b) = rest
    cid = lax.axis_index("c")
    sid = lax.axis_index("s")
    wid = sid * NC + cid

    pltpu.sync_copy(zeros_hbm, rows_a)
    if with_deg:
      pltpu.sync_copy(ones_hbm, ones_v)
      pltpu.sync_copy(zeros_hbm.at[0, :], zcol_v)
    for k in range(ROWCH_PER_S):
      r = sid + k * NS
      pltpu.sync_copy(rows_a, acc_sh.at[pl.ds(r * EC, EC), :])
      if with_deg:
        pltpu.sync_copy(zcol_v, deg_sh.at[pl.ds(r * EC, EC)])

    plsc.subcore_barrier()

    def fire(k, rows, sem):
      pltpu.async_copy(y_hbm.at[src_v.at[k]], rows, sem)

    def drain(rows, sem):
      pltpu.make_async_copy(y_hbm.at[src_v.at[0]], rows, sem).wait()

    def scat(k, rows):
      pltpu.sync_copy(rows, acc_sh.at[dst_v.at[k]], add=True)
      if with_deg:
        pltpu.sync_copy(ones_v, deg_sh.at[dst_v.at[k]], add=True)

    base = wid * CHUNKS_PER_W
    for half in range(2):
      pltpu.sync_copy(
          src_hbm.at[pl.ds(base + half * HALF_W, HALF_W), :], src_v)
      pltpu.sync_copy(
          dst_hbm.at[pl.ds(base + half * HALF_W, HALF_W), :], dst_v)
      fire(0, rows_a, sem_a)

      def edge_step(j, carry):
        k = j * 2
        fire(k + 1, rows_b, sem_b)
        drain(rows_a, sem_a)
        scat(k, rows_a)

        @pl.when(j < HALF_W // 2 - 1)
        def _():
          fire(k + 2, rows_a, sem_a)

        drain(rows_b, sem_b)
        scat(k + 1, rows_b)
        return carry

      lax.fori_loop(0, HALF_W // 2, edge_step, 0)
    plsc.subcore_barrier()

    for k in range(ROWCH_PER_S):
      r = sid + k * NS
      pltpu.sync_copy(acc_sh.at[pl.ds(r * EC, EC), :], rows_a)
      pltpu.sync_copy(rows_a, out_hbm.at[cid, pl.ds(r * EC, EC), :])
      if with_deg:
        pltpu.sync_copy(deg_sh.at[pl.ds(r * EC, EC)], ones_v)
        pltpu.sync_copy(ones_v, deg_hbm.at[cid, pl.ds(r * EC, EC)])

  return pl.kernel(
      body,
      out_type=tuple(out_type) if with_deg else out_type[0],
      mesh=mesh,
      scratch_types=scratch,
  )


def _lin_relu_kernel(x_ref, w_ref, b_ref, o_ref):
  o_ref[...] = jax.nn.relu(
      jnp.dot(x_ref[...], w_ref[...], preferred_element_type=jnp.float32)
      + b_ref[...])


def _node_linear(x, w, b):
  return pl.pallas_call(
      _lin_relu_kernel,
      grid=(GRID,),
      in_specs=[
          pl.BlockSpec((BLK, D), lambda i: (i, 0)),
          pl.BlockSpec((D, D), lambda i: (0, 0)),
          pl.BlockSpec((1, D), lambda i: (0, 0)),
      ],
      out_specs=pl.BlockSpec((BLK, D), lambda i: (i, 0)),
      out_shape=jax.ShapeDtypeStruct((N_PAD, D), jnp.float32),
  )(x, w, b.reshape(1, D))


def _combine_block(p_ref, deg_ref, x_ref, wt_ref, wb_ref, b_ref):
  psum = p_ref[0] + p_ref[1]
  dsum = deg_ref[0, 0, 0, :] + deg_ref[1, 0, 0, :]
  inv = 1.0 / jnp.maximum(dsum, 1.0)
  aggr = psum * inv[:, None]
  h = jax.nn.relu(
      jnp.dot(aggr, wt_ref[...], preferred_element_type=jnp.float32)
      + jnp.dot(x_ref[...], wb_ref[...], preferred_element_type=jnp.float32)
      + b_ref[...])
  nrm = jnp.sqrt(jnp.sum(h * h, axis=1, keepdims=True))
  return h / jnp.maximum(nrm, 1e-12)


def _combine2_kernel(p_ref, deg_ref, x_ref, wt_ref, wb_ref, b_ref,
                     w2_ref, b2_ref, h_ref, y2_ref):
  h = _combine_block(p_ref, deg_ref, x_ref, wt_ref, wb_ref, b_ref)
  h_ref[...] = h
  y2_ref[...] = jax.nn.relu(
      jnp.dot(h, w2_ref[...], preferred_element_type=jnp.float32)
      + b2_ref[...])


def _combine_and_next(p, deg, x, w_agg, b_agg, w_lin2, b_lin2):
  wt, wb = w_agg[:D], w_agg[D:]
  return pl.pallas_call(
      _combine2_kernel,
      grid=(GRID,),
      in_specs=[
          pl.BlockSpec((NC, BLK, D), lambda i: (0, i, 0)),
          pl.BlockSpec((NC, 1, 1, BLK), lambda i: (0, i, 0, 0)),
          pl.BlockSpec((BLK, D), lambda i: (i, 0)),
          pl.BlockSpec((D, D), lambda i: (0, 0)),
          pl.BlockSpec((D, D), lambda i: (0, 0)),
          pl.BlockSpec((1, D), lambda i: (0, 0)),
          pl.BlockSpec((D, D), lambda i: (0, 0)),
          pl.BlockSpec((1, D), lambda i: (0, 0)),
      ],
      out_specs=[
          pl.BlockSpec((BLK, D), lambda i: (i, 0)),
          pl.BlockSpec((BLK, D), lambda i: (i, 0)),
      ],
      out_shape=[
          jax.ShapeDtypeStruct((N_PAD, D), jnp.float32),
          jax.ShapeDtypeStruct((N_PAD, D), jnp.float32),
      ],
  )(p, deg.reshape(NC, GRID, 1, BLK), x, wt, wb, b_agg.reshape(1, D),
    w_lin2, b_lin2.reshape(1, D))


def _final_kernel(p_ref, deg_ref, x_ref, wt_ref, wb_ref, b_ref,
                  batch_ref, wp1_ref, bp1_ref, wp2_ref, bp2_ref,
                  o_ref, pool_ref):
  i = pl.program_id(0)
  h = _combine_block(p_ref, deg_ref, x_ref, wt_ref, wb_ref, b_ref)
  ids_col = batch_ref[0]

  @pl.when(i == 0)
  def _init():
    pool_ref[...] = jnp.full((N_GRAPHS, D), -jnp.inf, jnp.float32)

  lo = batch_ref[0, 0, 0]
  hi = batch_ref[0, BLK - 1, 0]
  neg = jnp.float32(-jnp.inf)
  for g in range(N_GRAPHS):
    @pl.when(jnp.logical_and(g >= lo, g <= hi))
    def _upd():
      m = ids_col == g
      mx = jnp.max(jnp.where(m, h, neg), axis=0)
      pool_ref[g, :] = jnp.maximum(pool_ref[g, :], mx)

  @pl.when(i == GRID - 1)
  def _fin():
    pooled = pool_ref[...]
    o1 = jnp.dot(pooled, wp1_ref[...],
                 preferred_element_type=jnp.float32) + bp1_ref[...]
    o2 = jnp.dot(o1, wp2_ref[...],
                 preferred_element_type=jnp.float32) + bp2_ref[...]
    z = o2 - jnp.max(o2, axis=1, keepdims=True)
    o_ref[...] = z - jnp.log(jnp.sum(jnp.exp(z), axis=1, keepdims=True))


def _final_stage(q, deg, h1, w_agg, b_agg, batch_col, wp1, bp1, wp2, bp2):
  wt, wb = w_agg[:D], w_agg[D:]
  return pl.pallas_call(
      _final_kernel,
      grid=(GRID,),
      in_specs=[
          pl.BlockSpec((NC, BLK, D), lambda i: (0, i, 0)),
          pl.BlockSpec((NC, 1, 1, BLK), lambda i: (0, i, 0, 0)),
          pl.BlockSpec((BLK, D), lambda i: (i, 0)),
          pl.BlockSpec((D, D), lambda i: (0, 0)),
          pl.BlockSpec((D, D), lambda i: (0, 0)),
          pl.BlockSpec((1, D), lambda i: (0, 0)),
          pl.BlockSpec((1, BLK, 1), lambda i: (i, 0, 0)),
          pl.BlockSpec((D, D), lambda i: (0, 0)),
          pl.BlockSpec((1, D), lambda i: (0, 0)),
          pl.BlockSpec((D, D_OUT), lambda i: (0, 0)),
          pl.BlockSpec((1, D_OUT), lambda i: (0, 0)),
      ],
      out_specs=pl.BlockSpec((N_GRAPHS, D_OUT), lambda i: (0, 0)),
      out_shape=jax.ShapeDtypeStruct((N_GRAPHS, D_OUT), jnp.float32),
      scratch_shapes=[pltpu.VMEM((N_GRAPHS, D), jnp.float32)],
  )(q, deg.reshape(NC, GRID, 1, BLK), h1, wt, wb, b_agg.reshape(1, D),
    batch_col, wp1, bp1.reshape(1, D),
    wp2, bp2.reshape(1, D_OUT))


def kernel(x, edge_index, batch,
           W_lin1, b_lin1, W_agg1, b_agg1,
           W_lin2, b_lin2, W_agg2, b_agg2,
           W_post1, b_post1, W_post2, b_post2):
  batch_col = jnp.pad(batch, (0, N_PAD - N_NODES),
                      constant_values=N_GRAPHS).reshape(GRID, BLK, 1)
  pad_iota = jnp.arange(E_PAD - N_EDGES, dtype=jnp.int32)
  src2d = jnp.concatenate(
      [edge_index[0], pad_iota % N_PAD]).reshape(N_CHUNKS, EC)
  dst2d = jnp.concatenate(
      [edge_index[1],
       N_NODES + pad_iota % (N_PAD - N_NODES)]).reshape(N_CHUNKS, EC)
  zeros = jnp.zeros((EC, D), jnp.float32)
  ones = jnp.ones((EC,), jnp.float32)

  y1 = _node_linear(x, W_lin1, b_lin1)
  p1, deg = _sc_segment_pass(True)(y1, src2d, dst2d, zeros, ones)
  h1, y2 = _combine_and_next(p1, deg, x, W_agg1, b_agg1, W_lin2, b_lin2)
  p2 = _sc_segment_pass(False)(y2, src2d, dst2d, zeros, ones)
  return _final_stage(p2, deg, h1, W_agg2, b_agg2, batch_col,
                      W_post1, b_post1, W_post2, b_post2)


# speedup vs baseline: 1.0088x; 1.0088x over previous
import functools

import jax
import jax.numpy as jnp
from jax import lax
from jax.experimental import pallas as pl
from jax.experimental.pallas import tpu as pltpu
from jax.experimental.pallas import tpu_sc as plsc

N_NODES = 10000
N_EDGES = 320000
D = 128
N_GRAPHS = 64
D_OUT = 16

NC = 2
NS = 16
NW = NC * NS
EC = 128

N_PAD = 10240
N_ROWCH = N_PAD // EC
ROWCH_PER_S = N_ROWCH // NS
E_PAD = 327680
N_CHUNKS = E_PAD // EC
CHUNKS_PER_W = N_CHUNKS // NW
HALF_W = CHUNKS_PER_W // 2

BLK = 1024
GRID = N_PAD // BLK


def _sc_segment_pass(with_deg: bool):
  mesh = plsc.VectorSubcoreMesh(core_axis_name="c", subcore_axis_name="s")
  out_type = [jax.ShapeDtypeStruct((NC, N_PAD, D), jnp.float32)]
  scratch = [
      pltpu.VMEM((HALF_W, EC), jnp.int32),
      pltpu.VMEM((HALF_W, EC), jnp.int32),
      pltpu.VMEM((EC, D), jnp.float32),
      pltpu.VMEM((EC, D), jnp.float32),
      pltpu.VMEM_SHARED((N_PAD, D), jnp.float32),
      pltpu.SemaphoreType.DMA,
      pltpu.SemaphoreType.DMA,
  ]
  if with_deg:
    out_type.append(jax.ShapeDtypeStruct((NC, N_PAD), jnp.float32))
    scratch += [
        pltpu.VMEM((EC,), jnp.float32),
        pltpu.VMEM((EC,), jnp.float32),
        pltpu.VMEM_SHARED((N_PAD,), jnp.float32),
    ]

  def body(y_hbm, src_hbm, dst_hbm, zeros_hbm, ones_hbm, *rest):
    if with_deg:
      (out_hbm, deg_hbm, src_v, dst_v, rows_a, rows_b, acc_sh,
       sem_a, sem_b, ones_v, zcol_v, deg_sh) = rest
    else:
      (out_hbm, src_v, dst_v, rows_a, rows_b, acc_sh, sem_a, sem_b) = rest
    cid = lax.axis_index("c")
    sid = lax.axis_index("s")
    wid = sid * NC + cid

    pltpu.sync_copy(zeros_hbm, rows_a)
    if with_deg:
      pltpu.sync_copy(ones_hbm, ones_v)
      pltpu.sync_copy(zeros_hbm.at[0, :], zcol_v)
    for k in range(ROWCH_PER_S):
      r = sid + k * NS
      pltpu.sync_copy(rows_a, acc_sh.at[pl.ds(r * EC, EC), :])
      if with_deg:
        pltpu.sync_copy(zcol_v, deg_sh.at[pl.ds(r * EC, EC)])

    plsc.subcore_barrier()

    def fire(k, rows, sem):
      pltpu.async_copy(y_hbm.at[src_v.at[k]], rows, sem)

    def drain(rows, sem):
      pltpu.make_async_copy(y_hbm.at[src_v.at[0]], rows, sem).wait()

    def scat(k, rows):
      pltpu.sync_copy(rows, acc_sh.at[dst_v.at[k]], add=True)
      if with_deg:
        pltpu.sync_copy(ones_v, deg_sh.at[dst_v.at[k]], add=True)

    base = wid * CHUNKS_PER_W
    for half in range(2):
      pltpu.sync_copy(
          src_hbm.at[pl.ds(base + half * HALF_W, HALF_W), :], src_v)
      pltpu.sync_copy(
          dst_hbm.at[pl.ds(base + half * HALF_W, HALF_W), :], dst_v)
      fire(0, rows_a, sem_a)

      def edge_step(j, carry):
        k = j * 2
        fire(k + 1, rows_b, sem_b)
        drain(rows_a, sem_a)
        scat(k, rows_a)

        @pl.when(j < HALF_W // 2 - 1)
        def _():
          fire(k + 2, rows_a, sem_a)

        drain(rows_b, sem_b)
        scat(k + 1, rows_b)
        return carry

      lax.fori_loop(0, HALF_W // 2, edge_step, 0)
    plsc.subcore_barrier()

    for k in range(ROWCH_PER_S):
      r = sid + k * NS
      pltpu.sync_copy(acc_sh.at[pl.ds(r * EC, EC), :], rows_a)
      pltpu.sync_copy(rows_a, out_hbm.at[cid, pl.ds(r * EC, EC), :])
      if with_deg:
        pltpu.sync_copy(deg_sh.at[pl.ds(r * EC, EC)], ones_v)
        pltpu.sync_copy(ones_v, deg_hbm.at[cid, pl.ds(r * EC, EC)])

  return pl.kernel(
      body,
      out_type=tuple(out_type) if with_deg else out_type[0],
      mesh=mesh,
      scratch_types=scratch,
  )


def _lin_relu_kernel(x_ref, w_ref, b_ref, o_ref):
  o_ref[...] = jax.nn.relu(
      jnp.dot(x_ref[...], w_ref[...], preferred_element_type=jnp.float32)
      + b_ref[...])


def _node_linear(x, w, b):
  return pl.pallas_call(
      _lin_relu_kernel,
      grid=(GRID,),
      in_specs=[
          pl.BlockSpec((BLK, D), lambda i: (i, 0)),
          pl.BlockSpec((D, D), lambda i: (0, 0)),
          pl.BlockSpec((1, D), lambda i: (0, 0)),
      ],
      out_specs=pl.BlockSpec((BLK, D), lambda i: (i, 0)),
      out_shape=jax.ShapeDtypeStruct((N_PAD, D), jnp.float32),
  )(x, w, b.reshape(1, D))


def _combine_block(p_ref, deg_ref, x_ref, wt_ref, wb_ref, b_ref):
  psum = p_ref[0] + p_ref[1]
  dsum = deg_ref[0, 0, 0, :] + deg_ref[1, 0, 0, :]
  inv = 1.0 / jnp.maximum(dsum, 1.0)
  aggr = psum * inv[:, None]
  h = jax.nn.relu(
      jnp.dot(aggr, wt_ref[...], preferred_element_type=jnp.float32)
      + jnp.dot(x_ref[...], wb_ref[...], preferred_element_type=jnp.float32)
      + b_ref[...])
  nrm = jnp.sqrt(jnp.sum(h * h, axis=1, keepdims=True))
  return h / jnp.maximum(nrm, 1e-12)


def _combine2_kernel(p_ref, deg_ref, x_ref, wt_ref, wb_ref, b_ref,
                     w2_ref, b2_ref, h_ref, y2_ref):
  h = _combine_block(p_ref, deg_ref, x_ref, wt_ref, wb_ref, b_ref)
  h_ref[...] = h
  y2_ref[...] = jax.nn.relu(
      jnp.dot(h, w2_ref[...], preferred_element_type=jnp.float32)
      + b2_ref[...])


def _combine_and_next(p, deg, x, w_agg, b_agg, w_lin2, b_lin2):
  wt, wb = w_agg[:D], w_agg[D:]
  return pl.pallas_call(
      _combine2_kernel,
      grid=(GRID,),
      in_specs=[
          pl.BlockSpec((NC, BLK, D), lambda i: (0, i, 0)),
          pl.BlockSpec((NC, 1, 1, BLK), lambda i: (0, i, 0, 0)),
          pl.BlockSpec((BLK, D), lambda i: (i, 0)),
          pl.BlockSpec((D, D), lambda i: (0, 0)),
          pl.BlockSpec((D, D), lambda i: (0, 0)),
          pl.BlockSpec((1, D), lambda i: (0, 0)),
          pl.BlockSpec((D, D), lambda i: (0, 0)),
          pl.BlockSpec((1, D), lambda i: (0, 0)),
      ],
      out_specs=[
          pl.BlockSpec((BLK, D), lambda i: (i, 0)),
          pl.BlockSpec((BLK, D), lambda i: (i, 0)),
      ],
      out_shape=[
          jax.ShapeDtypeStruct((N_PAD, D), jnp.float32),
          jax.ShapeDtypeStruct((N_PAD, D), jnp.float32),
      ],
  )(p, deg.reshape(NC, GRID, 1, BLK), x, wt, wb, b_agg.reshape(1, D),
    w_lin2, b_lin2.reshape(1, D))


def _final_kernel(p_ref, deg_ref, x_ref, wt_ref, wb_ref, b_ref,
                  batch_ref, wp1_ref, bp1_ref, wp2_ref, bp2_ref,
                  o_ref, pool_ref):
  i = pl.program_id(0)
  h = _combine_block(p_ref, deg_ref, x_ref, wt_ref, wb_ref, b_ref)
  ids_col = batch_ref[0]

  @pl.when(i == 0)
  def _init():
    pool_ref[...] = jnp.full((N_GRAPHS, D), -jnp.inf, jnp.float32)

  lo = batch_ref[0, 0, 0]
  hi = batch_ref[0, BLK - 1, 0]
  neg = jnp.float32(-jnp.inf)
  for g in range(N_GRAPHS):
    @pl.when(jnp.logical_and(g >= lo, g <= hi))
    def _upd():
      m = ids_col == g
      mx = jnp.max(jnp.where(m, h, neg), axis=0)
      pool_ref[g, :] = jnp.maximum(pool_ref[g, :], mx)

  @pl.when(i == GRID - 1)
  def _fin():
    pooled = pool_ref[...]
    o1 = jnp.dot(pooled, wp1_ref[...],
                 preferred_element_type=jnp.float32) + bp1_ref[...]
    o2 = jnp.dot(o1, wp2_ref[...],
                 preferred_element_type=jnp.float32) + bp2_ref[...]
    z = o2 - jnp.max(o2, axis=1, keepdims=True)
    o_ref[...] = z - jnp.log(jnp.sum(jnp.exp(z), axis=1, keepdims=True))


def _final_stage(q, deg, h1, w_agg, b_agg, batch_col, wp1, bp1, wp2, bp2):
  wt, wb = w_agg[:D], w_agg[D:]
  return pl.pallas_call(
      _final_kernel,
      grid=(GRID,),
      in_specs=[
          pl.BlockSpec((NC, BLK, D), lambda i: (0, i, 0)),
          pl.BlockSpec((NC, 1, 1, BLK), lambda i: (0, i, 0, 0)),
          pl.BlockSpec((BLK, D), lambda i: (i, 0)),
          pl.BlockSpec((D, D), lambda i: (0, 0)),
          pl.BlockSpec((D, D), lambda i: (0, 0)),
          pl.BlockSpec((1, D), lambda i: (0, 0)),
          pl.BlockSpec((1, BLK, 1), lambda i: (i, 0, 0)),
          pl.BlockSpec((D, D), lambda i: (0, 0)),
          pl.BlockSpec((1, D), lambda i: (0, 0)),
          pl.BlockSpec((D, D_OUT), lambda i: (0, 0)),
          pl.BlockSpec((1, D_OUT), lambda i: (0, 0)),
      ],
      out_specs=pl.BlockSpec((N_GRAPHS, D_OUT), lambda i: (0, 0)),
      out_shape=jax.ShapeDtypeStruct((N_GRAPHS, D_OUT), jnp.float32),
      scratch_shapes=[pltpu.VMEM((N_GRAPHS, D), jnp.float32)],
  )(q, deg.reshape(NC, GRID, 1, BLK), h1, wt, wb, b_agg.reshape(1, D),
    batch_col, wp1, bp1.reshape(1, D),
    wp2, bp2.reshape(1, D_OUT))


def kernel(x, edge_index, batch,
           W_lin1, b_lin1, W_agg1, b_agg1,
           W_lin2, b_lin2, W_agg2, b_agg2,
           W_post1, b_post1, W_post2, b_post2):
  batch_col = jnp.pad(batch, (0, N_PAD - N_NODES),
                      constant_values=N_GRAPHS).reshape(GRID, BLK, 1)
  pad_iota = jnp.arange(E_PAD - N_EDGES, dtype=jnp.int32)
  src2d = jnp.concatenate(
      [edge_index[0], pad_iota % N_PAD]).reshape(N_CHUNKS, EC)
  dst2d = jnp.concatenate(
      [edge_index[1],
       N_NODES + pad_iota % (N_PAD - N_NODES)]).reshape(N_CHUNKS, EC)
  zeros = jnp.zeros((EC, D), jnp.float32)
  ones = jnp.ones((EC,), jnp.float32)

  y1 = _node_linear(x, W_lin1, b_lin1)
  p1, deg = _sc_segment_pass(True)(y1, src2d, dst2d, zeros, ones)
  h1, y2 = _combine_and_next(p1, deg, x, W_agg1, b_agg1, W_lin2, b_lin2)
  p2 = _sc_segment_pass(False)(y2, src2d, dst2d, zeros, ones)
  return _final_stage(p2, deg, h1, W_agg2, b_agg2, batch_col,
                      W_post1, b_post1, W_post2, b_post2)


# speedup vs baseline: 1.0296x; 1.0207x over previous
import functools

import jax
import jax.numpy as jnp
from jax import lax
from jax.experimental import pallas as pl
from jax.experimental.pallas import tpu as pltpu
from jax.experimental.pallas import tpu_sc as plsc

N_NODES = 10000
N_EDGES = 320000
D = 128
N_GRAPHS = 64
D_OUT = 16

NC = 2
NS = 16
NW = NC * NS
EC = 128

N_PAD = 10240
N_ROWCH = N_PAD // EC
ROWCH_PER_S = N_ROWCH // NS
E_PAD = 327680
N_CHUNKS = E_PAD // EC
CHUNKS_PER_W = N_CHUNKS // NW
HALF_W = CHUNKS_PER_W // 2

BLK = 1024
GRID = N_PAD // BLK


def _sc_segment_pass(with_deg: bool):
  mesh = plsc.VectorSubcoreMesh(core_axis_name="c", subcore_axis_name="s")
  out_type = [jax.ShapeDtypeStruct((NC, N_PAD, D), jnp.float32)]
  scratch = [
      pltpu.VMEM((HALF_W, EC), jnp.int32),
      pltpu.VMEM((HALF_W, EC), jnp.int32),
      pltpu.VMEM((EC, D), jnp.float32),
      pltpu.VMEM((EC, D), jnp.float32),
      pltpu.VMEM_SHARED((N_PAD, D), jnp.float32),
      pltpu.SemaphoreType.DMA,
      pltpu.SemaphoreType.DMA,
  ]
  if with_deg:
    out_type.append(jax.ShapeDtypeStruct((NC, N_PAD), jnp.float32))
    scratch += [
        pltpu.VMEM((EC,), jnp.float32),
        pltpu.VMEM((EC,), jnp.float32),
        pltpu.VMEM_SHARED((N_PAD,), jnp.float32),
    ]

  def body(y_hbm, src_hbm, dst_hbm, zeros_hbm, ones_hbm, *rest):
    if with_deg:
      (out_hbm, deg_hbm, src_v, dst_v, rows_a, rows_b, acc_sh,
       sem_a, sem_b, ones_v, zcol_v, deg_sh) = rest
    else:
      (out_hbm, src_v, dst_v, rows_a, rows_b, acc_sh, sem_a, sem_b) = rest
    cid = lax.axis_index("c")
    sid = lax.axis_index("s")
    wid = sid * NC + cid

    pltpu.sync_copy(zeros_hbm, rows_b)
    if with_deg:
      pltpu.sync_copy(ones_hbm, ones_v)
      pltpu.sync_copy(zeros_hbm.at[0, :], zcol_v)
    base = wid * CHUNKS_PER_W
    pltpu.sync_copy(src_hbm.at[pl.ds(base, HALF_W), :], src_v)
    pltpu.sync_copy(dst_hbm.at[pl.ds(base, HALF_W), :], dst_v)
    pltpu.async_copy(y_hbm.at[src_v.at[0]], rows_a, sem_a)
    for k in range(ROWCH_PER_S):
      r = sid + k * NS
      pltpu.sync_copy(rows_b, acc_sh.at[pl.ds(r * EC, EC), :])
      if with_deg:
        pltpu.sync_copy(zcol_v, deg_sh.at[pl.ds(r * EC, EC)])

    plsc.subcore_barrier()

    def fire(k, rows, sem):
      pltpu.async_copy(y_hbm.at[src_v.at[k]], rows, sem)

    def drain(rows, sem):
      pltpu.make_async_copy(y_hbm.at[src_v.at[0]], rows, sem).wait()

    def scat(k, rows):
      pltpu.sync_copy(rows, acc_sh.at[dst_v.at[k]], add=True)
      if with_deg:
        pltpu.sync_copy(ones_v, deg_sh.at[dst_v.at[k]], add=True)

    for half in range(2):
      if half > 0:
        pltpu.sync_copy(
            src_hbm.at[pl.ds(base + half * HALF_W, HALF_W), :], src_v)
        pltpu.sync_copy(
            dst_hbm.at[pl.ds(base + half * HALF_W, HALF_W), :], dst_v)
        fire(0, rows_a, sem_a)

      def edge_step(j, carry):
        k = j * 2
        fire(k + 1, rows_b, sem_b)
        drain(rows_a, sem_a)
        scat(k, rows_a)

        @pl.when(j < HALF_W // 2 - 1)
        def _():
          fire(k + 2, rows_a, sem_a)

        drain(rows_b, sem_b)
        scat(k + 1, rows_b)
        return carry

      lax.fori_loop(0, HALF_W // 2, edge_step, 0)
    plsc.subcore_barrier()

    for k in range(ROWCH_PER_S):
      r = sid + k * NS
      buf, sem = (rows_a, sem_a) if k % 2 == 0 else (rows_b, sem_b)
      if k >= 2:
        pltpu.make_async_copy(
            buf, out_hbm.at[cid, pl.ds(r * EC, EC), :], sem).wait()
      pltpu.sync_copy(acc_sh.at[pl.ds(r * EC, EC), :], buf)
      pltpu.async_copy(buf, out_hbm.at[cid, pl.ds(r * EC, EC), :], sem)
      if with_deg:
        pltpu.sync_copy(deg_sh.at[pl.ds(r * EC, EC)], ones_v)
        pltpu.sync_copy(ones_v, deg_hbm.at[cid, pl.ds(r * EC, EC)])
    for k in (ROWCH_PER_S - 2, ROWCH_PER_S - 1):
      r = sid + k * NS
      buf, sem = (rows_a, sem_a) if k % 2 == 0 else (rows_b, sem_b)
      pltpu.make_async_copy(
          buf, out_hbm.at[cid, pl.ds(r * EC, EC), :], sem).wait()

  return pl.kernel(
      body,
      out_type=tuple(out_type) if with_deg else out_type[0],
      mesh=mesh,
      scratch_types=scratch,
  )


def _lin_relu_kernel(x_ref, w_ref, b_ref, o_ref):
  o_ref[...] = jax.nn.relu(
      jnp.dot(x_ref[...], w_ref[...], preferred_element_type=jnp.float32)
      + b_ref[...])


def _node_linear(x, w, b):
  return pl.pallas_call(
      _lin_relu_kernel,
      grid=(GRID,),
      in_specs=[
          pl.BlockSpec((BLK, D), lambda i: (i, 0)),
          pl.BlockSpec((D, D), lambda i: (0, 0)),
          pl.BlockSpec((1, D), lambda i: (0, 0)),
      ],
      out_specs=pl.BlockSpec((BLK, D), lambda i: (i, 0)),
      out_shape=jax.ShapeDtypeStruct((N_PAD, D), jnp.float32),
  )(x, w, b.reshape(1, D))


def _combine_block(p_ref, deg_ref, x_ref, wt_ref, wb_ref, b_ref):
  psum = p_ref[0] + p_ref[1]
  dsum = deg_ref[0, 0, 0, :] + deg_ref[1, 0, 0, :]
  inv = 1.0 / jnp.maximum(dsum, 1.0)
  aggr = psum * inv[:, None]
  h = jax.nn.relu(
      jnp.dot(aggr, wt_ref[...], preferred_element_type=jnp.float32)
      + jnp.dot(x_ref[...], wb_ref[...], preferred_element_type=jnp.float32)
      + b_ref[...])
  nrm = jnp.sqrt(jnp.sum(h * h, axis=1, keepdims=True))
  return h / jnp.maximum(nrm, 1e-12)


def _combine2_kernel(p_ref, deg_ref, x_ref, wt_ref, wb_ref, b_ref,
                     w2_ref, b2_ref, h_ref, y2_ref):
  h = _combine_block(p_ref, deg_ref, x_ref, wt_ref, wb_ref, b_ref)
  h_ref[...] = h
  y2_ref[...] = jax.nn.relu(
      jnp.dot(h, w2_ref[...], preferred_element_type=jnp.float32)
      + b2_ref[...])


def _combine_and_next(p, deg, x, w_agg, b_agg, w_lin2, b_lin2):
  wt, wb = w_agg[:D], w_agg[D:]
  return pl.pallas_call(
      _combine2_kernel,
      grid=(GRID,),
      in_specs=[
          pl.BlockSpec((NC, BLK, D), lambda i: (0, i, 0)),
          pl.BlockSpec((NC, 1, 1, BLK), lambda i: (0, i, 0, 0)),
          pl.BlockSpec((BLK, D), lambda i: (i, 0)),
          pl.BlockSpec((D, D), lambda i: (0, 0)),
          pl.BlockSpec((D, D), lambda i: (0, 0)),
          pl.BlockSpec((1, D), lambda i: (0, 0)),
          pl.BlockSpec((D, D), lambda i: (0, 0)),
          pl.BlockSpec((1, D), lambda i: (0, 0)),
      ],
      out_specs=[
          pl.BlockSpec((BLK, D), lambda i: (i, 0)),
          pl.BlockSpec((BLK, D), lambda i: (i, 0)),
      ],
      out_shape=[
          jax.ShapeDtypeStruct((N_PAD, D), jnp.float32),
          jax.ShapeDtypeStruct((N_PAD, D), jnp.float32),
      ],
  )(p, deg.reshape(NC, GRID, 1, BLK), x, wt, wb, b_agg.reshape(1, D),
    w_lin2, b_lin2.reshape(1, D))


def _final_kernel(p_ref, deg_ref, x_ref, wt_ref, wb_ref, b_ref,
                  batch_ref, wp1_ref, bp1_ref, wp2_ref, bp2_ref,
                  o_ref, pool_ref):
  i = pl.program_id(0)
  h = _combine_block(p_ref, deg_ref, x_ref, wt_ref, wb_ref, b_ref)
  ids_col = batch_ref[0]

  @pl.when(i == 0)
  def _init():
    pool_ref[...] = jnp.full((N_GRAPHS, D), -jnp.inf, jnp.float32)

  lo = batch_ref[0, 0, 0]
  hi = batch_ref[0, BLK - 1, 0]
  neg = jnp.float32(-jnp.inf)
  for g in range(N_GRAPHS):
    @pl.when(jnp.logical_and(g >= lo, g <= hi))
    def _upd():
      m = ids_col == g
      mx = jnp.max(jnp.where(m, h, neg), axis=0)
      pool_ref[g, :] = jnp.maximum(pool_ref[g, :], mx)

  @pl.when(i == GRID - 1)
  def _fin():
    pooled = pool_ref[...]
    o1 = jnp.dot(pooled, wp1_ref[...],
                 preferred_element_type=jnp.float32) + bp1_ref[...]
    o2 = jnp.dot(o1, wp2_ref[...],
                 preferred_element_type=jnp.float32) + bp2_ref[...]
    z = o2 - jnp.max(o2, axis=1, keepdims=True)
    o_ref[...] = z - jnp.log(jnp.sum(jnp.exp(z), axis=1, keepdims=True))


def _final_stage(q, deg, h1, w_agg, b_agg, batch_col, wp1, bp1, wp2, bp2):
  wt, wb = w_agg[:D], w_agg[D:]
  return pl.pallas_call(
      _final_kernel,
      grid=(GRID,),
      in_specs=[
          pl.BlockSpec((NC, BLK, D), lambda i: (0, i, 0)),
          pl.BlockSpec((NC, 1, 1, BLK), lambda i: (0, i, 0, 0)),
          pl.BlockSpec((BLK, D), lambda i: (i, 0)),
          pl.BlockSpec((D, D), lambda i: (0, 0)),
          pl.BlockSpec((D, D), lambda i: (0, 0)),
          pl.BlockSpec((1, D), lambda i: (0, 0)),
          pl.BlockSpec((1, BLK, 1), lambda i: (i, 0, 0)),
          pl.BlockSpec((D, D), lambda i: (0, 0)),
          pl.BlockSpec((1, D), lambda i: (0, 0)),
          pl.BlockSpec((D, D_OUT), lambda i: (0, 0)),
          pl.BlockSpec((1, D_OUT), lambda i: (0, 0)),
      ],
      out_specs=pl.BlockSpec((N_GRAPHS, D_OUT), lambda i: (0, 0)),
      out_shape=jax.ShapeDtypeStruct((N_GRAPHS, D_OUT), jnp.float32),
      scratch_shapes=[pltpu.VMEM((N_GRAPHS, D), jnp.float32)],
  )(q, deg.reshape(NC, GRID, 1, BLK), h1, wt, wb, b_agg.reshape(1, D),
    batch_col, wp1, bp1.reshape(1, D),
    wp2, bp2.reshape(1, D_OUT))


def kernel(x, edge_index, batch,
           W_lin1, b_lin1, W_agg1, b_agg1,
           W_lin2, b_lin2, W_agg2, b_agg2,
           W_post1, b_post1, W_post2, b_post2):
  batch_col = jnp.pad(batch, (0, N_PAD - N_NODES),
                      constant_values=N_GRAPHS).reshape(GRID, BLK, 1)
  pad_iota = jnp.arange(E_PAD - N_EDGES, dtype=jnp.int32)
  src2d = jnp.concatenate(
      [edge_index[0], pad_iota % N_PAD]).reshape(N_CHUNKS, EC)
  dst2d = jnp.concatenate(
      [edge_index[1],
       N_NODES + pad_iota % (N_PAD - N_NODES)]).reshape(N_CHUNKS, EC)
  zeros = jnp.zeros((EC, D), jnp.float32)
  ones = jnp.ones((EC,), jnp.float32)

  y1 = _node_linear(x, W_lin1, b_lin1)
  p1, deg = _sc_segment_pass(True)(y1, src2d, dst2d, zeros, ones)
  h1, y2 = _combine_and_next(p1, deg, x, W_agg1, b_agg1, W_lin2, b_lin2)
  p2 = _sc_segment_pass(False)(y2, src2d, dst2d, zeros, ones)
  return _final_stage(p2, deg, h1, W_agg2, b_agg2, batch_col,
                      W_post1, b_post1, W_post2, b_post2)
